# Initial kernel scaffold; baseline (speedup 1.0000x reference)
import jax
import jax.numpy as jnp
from jax.experimental import pallas as pl


def kernel(seq_a, edge_index, edge_weight, embedding, W1, b1, W2, b2):
    raise NotImplementedError("write your pallas kernel here")


# speedup vs baseline: 5.8593x; 5.8593x over previous
import dataclasses
import functools

import jax
import jax.numpy as jnp
from jax import lax
from jax.experimental import pallas as pl
from jax.experimental.pallas import tpu as pltpu
from jax.experimental.pallas import tpu_sc as plsc

N = 10000
D = 128
E = 320000

NC = 2
NS = 16
NW = NC * NS
C = 128
NCH = 80
EPW = NCH * C
E_PAD = NW * EPW
NPAD = 10240
RPW = NPAD // NS

_sc_mesh = plsc.VectorSubcoreMesh(core_axis_name="c", subcore_axis_name="s")

_sc_params = pltpu.CompilerParams()
if "needs_layout_passes" in pltpu.CompilerParams.__dataclass_fields__:
    _sc_params = dataclasses.replace(_sc_params, needs_layout_passes=False)


@functools.partial(
    pl.kernel,
    out_type=jax.ShapeDtypeStruct((NC, NPAD, D), jnp.float32),
    mesh=_sc_mesh,
    scratch_types=[
        pltpu.VMEM((NCH, C), jnp.int32),
        pltpu.VMEM((NCH, C), jnp.int32),
        pltpu.VMEM((NCH, C), jnp.float32),
        pltpu.VMEM((C, D), jnp.float32),
        pltpu.VMEM_SHARED((NPAD, D), jnp.float32),
        pltpu.SemaphoreType.DMA,
    ],
    compiler_params=_sc_params,
)
def spmm(h_hbm, src_hbm, dst_hbm, w_hbm, out_hbm,
         src_loc, dst_loc, w_loc, rows, acc, sem):
    cid = lax.axis_index("c")
    tid = lax.axis_index("s")
    wid = cid * NS + tid

    zero16 = jnp.zeros((16,), jnp.float32)

    @pl.loop(0, C)
    def _(r):
        for g in range(D // 16):
            rows[r, pl.ds(16 * g, 16)] = zero16

    @pl.loop(0, RPW // C)
    def _(z):
        pltpu.sync_copy(rows, acc.at[pl.ds(tid * RPW + z * C, C)])

    pltpu.sync_copy(src_hbm.at[pl.ds(wid * NCH, NCH)], src_loc)
    pltpu.sync_copy(dst_hbm.at[pl.ds(wid * NCH, NCH)], dst_loc)
    pltpu.sync_copy(w_hbm.at[pl.ds(wid * NCH, NCH)], w_loc)

    plsc.subcore_barrier()

    @pl.loop(0, NCH)
    def _(j):
        pltpu.async_copy(h_hbm.at[src_loc.at[j]], rows, sem).wait()

        @pl.loop(0, C)
        def _(e):
            ij = jnp.full((16,), j, jnp.int32)
            ie = jnp.full((16,), e, jnp.int32)
            ws = plsc.load_gather(w_loc, [ij, ie])
            for g in range(D // 16):
                sl = pl.ds(16 * g, 16)
                rows[e, sl] = rows[e, sl] * ws

        pltpu.sync_copy(rows, acc.at[dst_loc.at[j]], add=True)

    plsc.subcore_barrier()

    pltpu.sync_copy(acc.at[pl.ds(tid * RPW, RPW)],
                    out_hbm.at[cid].at[pl.ds(tid * RPW, RPW)])


def _mlp(emb, w1t, b1r, w2t, b2r):
    def body(x_ref, w1_ref, b1_ref, w2_ref, b2_ref, o_ref):
        x = x_ref[...]
        h = jnp.dot(x, w1_ref[...], preferred_element_type=jnp.float32,
                    precision=lax.Precision.HIGHEST) + b1_ref[...]
        h = jnp.where(h > 0, h, 0.1 * h)
        o_ref[...] = jnp.dot(h, w2_ref[...], preferred_element_type=jnp.float32,
                             precision=lax.Precision.HIGHEST) + b2_ref[...]

    return pl.pallas_call(
        body,
        grid=(5,),
        in_specs=[
            pl.BlockSpec((N // 5, D), lambda i: (i, 0)),
            pl.BlockSpec((D, D), lambda i: (0, 0)),
            pl.BlockSpec((1, D), lambda i: (0, 0)),
            pl.BlockSpec((D, D), lambda i: (0, 0)),
            pl.BlockSpec((1, D), lambda i: (0, 0)),
        ],
        out_specs=pl.BlockSpec((N // 5, D), lambda i: (i, 0)),
        out_shape=jax.ShapeDtypeStruct((N, D), jnp.float32),
    )(emb, w1t, b1r, w2t, b2r)


def _combine(p):
    def body(p_ref, o_ref):
        o_ref[...] = p_ref[0] + p_ref[1]

    blk = NPAD // 5
    return pl.pallas_call(
        body,
        grid=(5,),
        in_specs=[pl.BlockSpec((2, blk, D), lambda i: (0, i, 0))],
        out_specs=pl.BlockSpec((blk, D), lambda i: (i, 0)),
        out_shape=jax.ShapeDtypeStruct((NPAD, D), jnp.float32),
    )(p)


def kernel(seq_a, edge_index, edge_weight, embedding, W1, b1, W2, b2):
    h_a = _mlp(embedding, W1.T, b1.reshape(1, D), W2.T, b2.reshape(1, D))

    dst = edge_index[0]
    src = edge_index[1]
    pad = E_PAD - E
    pad_idx = jnp.arange(pad, dtype=jnp.int32)
    src_p = jnp.concatenate([src, pad_idx % N]).reshape(NW * NCH, C)
    dst_p = jnp.concatenate([dst, N + pad_idx % (NPAD - N)]).reshape(NW * NCH, C)
    w_p = jnp.concatenate([edge_weight, jnp.zeros((pad,), jnp.float32)]
                          ).reshape(NW * NCH, C)

    p1 = spmm(h_a, src_p, dst_p, w_p)
    m1 = _combine(p1)
    p2 = spmm(m1, src_p, dst_p, w_p)
    h_p = _combine(p2)
    return h_p[:N]
